# baseline (speedup 1.0000x reference)
import functools

import jax
import jax.numpy as jnp
from jax import lax
from jax.experimental import pallas as pl
from jax.experimental.pallas import tpu as pltpu
from jax.experimental.pallas import tpu_sc as plsc

CODEBOOK = 8192
DIM = 256
BATCH = 32
SEQ = 2048
EPS = 1e-5
L = 16
NC = 2
NS = 16
NW = NC * NS
DPW = DIM // NW


def _sc_decode(codes2, cluster_usage, embedding_sum):
  mesh = plsc.VectorSubcoreMesh(core_axis_name="c", subcore_axis_name="s")

  @functools.partial(
      pl.kernel,
      out_type=jax.ShapeDtypeStruct((BATCH, DIM, SEQ), jnp.float32),
      mesh=mesh,
      compiler_params=pltpu.CompilerParams(
          use_tc_tiling_on_sc=False, needs_layout_passes=False),
      scratch_types=[
          pltpu.VMEM((CODEBOOK, DPW), jnp.float32),
          pltpu.VMEM((CODEBOOK,), jnp.float32),
          pltpu.VMEM((SEQ,), jnp.int32),
          pltpu.VMEM((DPW, SEQ), jnp.float32),
      ],
  )
  def k(codes_hbm, usage_hbm, emb_hbm, out_hbm, cols_v, inv_v, idx_v, out_v):
    wid = lax.axis_index("s") * NC + lax.axis_index("c")
    dlo = pl.multiple_of(wid * DPW, DPW)

    pltpu.sync_copy(emb_hbm.at[:, pl.ds(dlo, DPW)], cols_v)
    pltpu.sync_copy(usage_hbm, inv_v)

    @pl.loop(0, CODEBOOK, step=L)
    def _(i):
      u = inv_v[pl.ds(i, L)]
      inv_v[pl.ds(i, L)] = 1.0 / jnp.maximum(u, EPS)

    @pl.loop(0, BATCH)
    def _(b):
      pltpu.sync_copy(codes_hbm.at[b], idx_v)

      @pl.loop(0, SEQ, step=L)
      def _(t):
        idx = idx_v[pl.ds(t, L)]
        r = plsc.load_gather(inv_v, [idx])
        for j in range(DPW):
          jj = jnp.full((L,), j, jnp.int32)
          g = plsc.load_gather(cols_v, [idx, jj])
          out_v[j, pl.ds(t, L)] = g * r

      pltpu.sync_copy(out_v, out_hbm.at[b, pl.ds(dlo, DPW), :])

  return k(codes2, cluster_usage, embedding_sum)


@jax.jit
def kernel(codes, cluster_usage, embedding_sum):
  codes2 = codes.reshape(BATCH, SEQ)
  return _sc_decode(codes2, cluster_usage, embedding_sum)


# speedup vs baseline: 2.4012x; 2.4012x over previous
import functools

import jax
import jax.numpy as jnp
from jax import lax
from jax.experimental import pallas as pl
from jax.experimental.pallas import tpu as pltpu
from jax.experimental.pallas import tpu_sc as plsc

CODEBOOK = 8192
DIM = 256
BATCH = 32
SEQ = 2048
EPS = 1e-5
L = 16
NC = 2
NS = 16
NW = NC * NS
DPW = DIM // NW


def _sc_decode(codes2, cluster_usage, embedding_sum):
  mesh = plsc.VectorSubcoreMesh(core_axis_name="c", subcore_axis_name="s")

  @functools.partial(
      pl.kernel,
      out_type=jax.ShapeDtypeStruct((BATCH, DIM, SEQ), jnp.float32),
      mesh=mesh,
      compiler_params=pltpu.CompilerParams(
          use_tc_tiling_on_sc=False, needs_layout_passes=False),
      scratch_types=[
          pltpu.VMEM((CODEBOOK, DPW), jnp.float32),
          pltpu.VMEM((CODEBOOK,), jnp.float32),
          pltpu.VMEM((2, SEQ), jnp.int32),
          pltpu.VMEM((2, DPW, SEQ), jnp.float32),
          pltpu.SemaphoreType.DMA,
          pltpu.SemaphoreType.DMA,
          pltpu.SemaphoreType.DMA,
          pltpu.SemaphoreType.DMA,
      ],
  )
  def k(codes_hbm, usage_hbm, emb_hbm, out_hbm, cols_v, inv_v, idx_v, out_v,
        isem0, isem1, osem0, osem1):
    isems = (isem0, isem1)
    osems = (osem0, osem1)
    wid = lax.axis_index("s") * NC + lax.axis_index("c")
    dlo = pl.multiple_of(wid * DPW, DPW)

    pltpu.sync_copy(emb_hbm.at[:, pl.ds(dlo, DPW)], cols_v)
    pltpu.sync_copy(usage_hbm, inv_v)

    @pl.loop(0, CODEBOOK, step=L)
    def _(i):
      u = inv_v[pl.ds(i, L)]
      inv_v[pl.ds(i, L)] = 1.0 / jnp.maximum(u, EPS)

    pltpu.async_copy(codes_hbm.at[0], idx_v.at[0], isems[0])

    def half(bb, h):
      ibuf = idx_v.at[h]
      obuf = out_v.at[h]
      pltpu.make_async_copy(codes_hbm.at[bb], ibuf, isems[h]).wait()

      @pl.when(bb + 1 < BATCH)
      def _():
        pltpu.async_copy(codes_hbm.at[bb + 1], idx_v.at[1 - h], isems[1 - h])

      @pl.when(bb >= 2)
      def _():
        pltpu.make_async_copy(
            obuf, out_hbm.at[bb, pl.ds(dlo, DPW), :], osems[h]).wait()

      @plsc.parallel_loop(0, SEQ, step=L)
      def _(t):
        idx = ibuf[pl.ds(t, L)]
        r = plsc.load_gather(inv_v, [idx])
        gs = [
            plsc.load_gather(cols_v, [idx, jnp.full((L,), j, jnp.int32)])
            for j in range(DPW)
        ]
        for j in range(DPW):
          obuf[j, pl.ds(t, L)] = gs[j] * r

      pltpu.async_copy(obuf, out_hbm.at[bb, pl.ds(dlo, DPW), :], osems[h])

    @pl.loop(0, BATCH, step=2)
    def _(b):
      half(b, 0)
      half(b + 1, 1)

    pltpu.make_async_copy(
        out_v.at[0], out_hbm.at[0, pl.ds(dlo, DPW), :], osems[0]).wait()
    pltpu.make_async_copy(
        out_v.at[1], out_hbm.at[0, pl.ds(dlo, DPW), :], osems[1]).wait()

  return k(codes2, cluster_usage, embedding_sum)


@jax.jit
def kernel(codes, cluster_usage, embedding_sum):
  codes2 = codes.reshape(BATCH, SEQ)
  return _sc_decode(codes2, cluster_usage, embedding_sum)


# speedup vs baseline: 4.3008x; 1.7911x over previous
import functools

import jax
import jax.numpy as jnp
from jax import lax
from jax.experimental import pallas as pl
from jax.experimental.pallas import tpu as pltpu
from jax.experimental.pallas import tpu_sc as plsc

CODEBOOK = 8192
DIM = 256
BATCH = 32
SEQ = 2048
EPS = 1e-5
L = 16
NC = 2
NS = 16
NW = NC * NS
DPW = DIM // NW
VBLK = 512


def _tc_scaled_transpose(usage2, emb):
  def body(u_ref, e_ref, o_ref):
    inv = 1.0 / jnp.maximum(u_ref[...], EPS)
    o_ref[...] = jnp.transpose(e_ref[...] * inv)

  return pl.pallas_call(
      body,
      grid=(CODEBOOK // VBLK,),
      in_specs=[
          pl.BlockSpec((VBLK, 1), lambda i: (i, 0)),
          pl.BlockSpec((VBLK, DIM), lambda i: (i, 0)),
      ],
      out_specs=pl.BlockSpec((DIM, VBLK), lambda i: (0, i)),
      out_shape=jax.ShapeDtypeStruct((DIM, CODEBOOK), jnp.float32),
  )(usage2, emb)


def _sc_decode(codes_flat, embt):
  mesh = plsc.VectorSubcoreMesh(core_axis_name="c", subcore_axis_name="s")

  @functools.partial(
      pl.kernel,
      out_type=jax.ShapeDtypeStruct((BATCH, DIM, SEQ), jnp.float32),
      mesh=mesh,
      compiler_params=pltpu.CompilerParams(needs_layout_passes=False),
      scratch_types=[
          pltpu.VMEM((DPW, CODEBOOK), jnp.float32),
          pltpu.VMEM((2, SEQ), jnp.int32),
          pltpu.VMEM((2, DPW, SEQ), jnp.float32),
          pltpu.SemaphoreType.DMA,
          pltpu.SemaphoreType.DMA,
          pltpu.SemaphoreType.DMA,
          pltpu.SemaphoreType.DMA,
      ],
  )
  def k(codes_hbm, embt_hbm, out_hbm, rows_v, idx_v, out_v,
        isem0, isem1, osem0, osem1):
    isems = (isem0, isem1)
    osems = (osem0, osem1)
    wid = lax.axis_index("s") * NC + lax.axis_index("c")
    dlo = pl.multiple_of(wid * DPW, DPW)

    pltpu.sync_copy(embt_hbm.at[pl.ds(dlo, DPW), :], rows_v)

    pltpu.async_copy(codes_hbm.at[pl.ds(0, SEQ)], idx_v.at[0], isems[0])

    def half(bb, h):
      ibuf = idx_v.at[h]
      obuf = out_v.at[h]
      pltpu.make_async_copy(
          codes_hbm.at[pl.ds(0, SEQ)], ibuf, isems[h]).wait()

      @pl.when(bb + 1 < BATCH)
      def _():
        pltpu.async_copy(
            codes_hbm.at[pl.ds((bb + 1) * SEQ, SEQ)], idx_v.at[1 - h],
            isems[1 - h])

      @pl.when(bb >= 2)
      def _():
        pltpu.make_async_copy(
            obuf, out_hbm.at[bb, pl.ds(dlo, DPW), :], osems[h]).wait()

      @plsc.parallel_loop(0, SEQ, step=L)
      def _(t):
        idx = idx_v[h, pl.ds(t, L)]
        gs = [
            plsc.load_gather(rows_v, [jnp.full((L,), j, jnp.int32), idx])
            for j in range(DPW)
        ]
        for j in range(DPW):
          out_v[h, j, pl.ds(t, L)] = gs[j]

      pltpu.async_copy(obuf, out_hbm.at[bb, pl.ds(dlo, DPW), :], osems[h])

    @pl.loop(0, BATCH, step=2)
    def _(b):
      half(b, 0)
      half(b + 1, 1)

    pltpu.make_async_copy(
        out_v.at[0], out_hbm.at[0, pl.ds(dlo, DPW), :], osems[0]).wait()
    pltpu.make_async_copy(
        out_v.at[1], out_hbm.at[0, pl.ds(dlo, DPW), :], osems[1]).wait()

  return k(codes_flat, embt)


@jax.jit
def kernel(codes, cluster_usage, embedding_sum):
  codes_flat = codes.reshape(BATCH * SEQ)
  usage2 = cluster_usage.reshape(CODEBOOK, 1)
  embt = _tc_scaled_transpose(usage2, embedding_sum)
  return _sc_decode(codes_flat, embt)
